# baseline (speedup 1.0000x reference)
import functools

import jax
import jax.numpy as jnp
from jax import lax
from jax.experimental import pallas as pl
from jax.experimental.pallas import tpu as pltpu
from jax.experimental.pallas import tpu_sc as plsc

_NC = 2
_NS = 16
_NW = _NC * _NS
_L = 16


def _dispatch_sc(x, gtok, pad_a):
    t, h = x.shape
    b_per_w = pad_a // _NW
    ch = b_per_w // 4
    mesh = plsc.VectorSubcoreMesh(core_axis_name="c", subcore_axis_name="s")

    @functools.partial(
        pl.kernel, mesh=mesh,
        out_type=jax.ShapeDtypeStruct((pad_a, h), jnp.float32),
        scratch_types=[
            pltpu.VMEM((b_per_w,), jnp.int32),
            pltpu.VMEM((ch, h), jnp.float32),
            pltpu.VMEM((ch, h), jnp.float32),
            pltpu.SemaphoreType.DMA,
            pltpu.SemaphoreType.DMA,
            pltpu.SemaphoreType.DMA,
            pltpu.SemaphoreType.DMA,
        ],
    )
    def k(x_hbm, idx_hbm, out_hbm, idx_v, rows0, rows1, sg0, sg1, sw0, sw1):
        wid = lax.axis_index("s") * _NC + lax.axis_index("c")
        base = wid * b_per_w
        pltpu.sync_copy(idx_hbm.at[pl.ds(base, b_per_w)], idx_v)
        g0 = pltpu.async_copy(x_hbm.at[idx_v.at[pl.ds(0, ch)]], rows0, sg0)
        g1 = pltpu.async_copy(x_hbm.at[idx_v.at[pl.ds(ch, ch)]], rows1, sg1)
        g0.wait()
        w0 = pltpu.async_copy(rows0, out_hbm.at[pl.ds(base, ch)], sw0)
        g1.wait()
        w1 = pltpu.async_copy(rows1, out_hbm.at[pl.ds(base + ch, ch)], sw1)
        w0.wait()
        g2 = pltpu.async_copy(
            x_hbm.at[idx_v.at[pl.ds(2 * ch, ch)]], rows0, sg0)
        w1.wait()
        g3 = pltpu.async_copy(
            x_hbm.at[idx_v.at[pl.ds(3 * ch, ch)]], rows1, sg1)
        g2.wait()
        w2 = pltpu.async_copy(rows0, out_hbm.at[pl.ds(base + 2 * ch, ch)], sw0)
        g3.wait()
        w3 = pltpu.async_copy(rows1, out_hbm.at[pl.ds(base + 3 * ch, ch)], sw1)
        w2.wait()
        w3.wait()

    return k(x, gtok)


def _combine_sc(dg, pos0, pos1):
    t = pos0.shape[0]
    h = dg.shape[1]
    t_per_w = t // _NW
    cht = min(32, t_per_w)
    mesh = plsc.VectorSubcoreMesh(core_axis_name="c", subcore_axis_name="s")

    @functools.partial(
        pl.kernel, mesh=mesh,
        out_type=jax.ShapeDtypeStruct((t, h), jnp.float32),
        scratch_types=[
            pltpu.VMEM((t_per_w,), jnp.int32),
            pltpu.VMEM((t_per_w,), jnp.int32),
            pltpu.VMEM((cht, h), jnp.float32),
            pltpu.VMEM((cht, h), jnp.float32),
            pltpu.SemaphoreType.DMA,
        ],
    )
    def k(dg_hbm, p0_hbm, p1_hbm, out_hbm, p0_v, p1_v, buf0, buf1, sem):
        wid = lax.axis_index("s") * _NC + lax.axis_index("c")
        base = wid * t_per_w
        pltpu.sync_copy(p0_hbm.at[pl.ds(base, t_per_w)], p0_v)
        pltpu.sync_copy(p1_hbm.at[pl.ds(base, t_per_w)], p1_v)

        @pl.loop(0, t_per_w // cht)
        def _(c):
            cp0 = pltpu.async_copy(
                dg_hbm.at[p0_v.at[pl.ds(c * cht, cht)]], buf0, sem)
            cp1 = pltpu.async_copy(
                dg_hbm.at[p1_v.at[pl.ds(c * cht, cht)]], buf1, sem)
            cp0.wait()
            cp1.wait()

            @pl.loop(0, cht)
            def _(r):
                @pl.loop(0, h // (4 * _L))
                def _(q):
                    for u in range(4):
                        slc = (pl.ds(r, 1), pl.ds(q * 4 * _L + u * _L, _L))
                        buf0.at[*slc][...] = (
                            buf0.at[*slc][...] + buf1.at[*slc][...])

            pltpu.sync_copy(buf0, out_hbm.at[pl.ds(base + c * cht, cht)])

    return k(dg, pos0, pos1)


def _grouped_gemm_body(te_ref, nt_ref, xg_ref, w13_ref, w2_ref, ws_ref,
                       dg_ref, w13bf_ref):
    i = pl.program_id(0)
    i_dim = w13_ref.shape[1] // 2
    te_i = te_ref[i]
    te_prev = te_ref[jnp.maximum(i - 1, 0)]
    new_expert = jnp.logical_or(i == 0, te_i != te_prev)

    @pl.when(jnp.logical_and(i < nt_ref[0], new_expert))
    def _():
        w13bf_ref[...] = w13_ref[0].astype(jnp.bfloat16)

    @pl.when(i < nt_ref[0])
    def _():
        x = xg_ref[...].astype(jnp.bfloat16)
        gate = jax.lax.dot_general(
            x, w13bf_ref[:i_dim],
            dimension_numbers=(((1,), (1,)), ((), ())),
            preferred_element_type=jnp.float32)
        up = jax.lax.dot_general(
            x, w13bf_ref[i_dim:],
            dimension_numbers=(((1,), (1,)), ((), ())),
            preferred_element_type=jnp.float32)
        act = (jax.nn.silu(gate) * up).astype(jnp.bfloat16)
        down = jax.lax.dot_general(
            act, w2_ref[0],
            dimension_numbers=(((1,), (1,)), ((), ())),
            preferred_element_type=jnp.float32)
        dg_ref[...] = down * ws_ref[0]


def _grouped_gemm(xg, w13_f32, w2_bf, wslot, tile_expert, nt_valid,
                  tile, maxtiles):
    pad_a, h = xg.shape
    e, i2, _ = w13_f32.shape
    i_dim = i2 // 2
    ws3 = wslot.reshape(maxtiles, tile, 1)
    grid_spec = pltpu.PrefetchScalarGridSpec(
        num_scalar_prefetch=2,
        grid=(maxtiles,),
        in_specs=[
            pl.BlockSpec((tile, h), lambda i, te, nt: (i, 0)),
            pl.BlockSpec((1, i2, h), lambda i, te, nt: (te[i], 0, 0)),
            pl.BlockSpec((1, h, i_dim), lambda i, te, nt: (te[i], 0, 0)),
            pl.BlockSpec((1, tile, 1), lambda i, te, nt: (i, 0, 0)),
        ],
        out_specs=pl.BlockSpec((tile, h), lambda i, te, nt: (i, 0)),
        scratch_shapes=[pltpu.VMEM((i2, h), jnp.bfloat16)],
    )
    return pl.pallas_call(
        _grouped_gemm_body,
        grid_spec=grid_spec,
        out_shape=jax.ShapeDtypeStruct((pad_a, h), jnp.float32),
        compiler_params=pltpu.CompilerParams(
            dimension_semantics=("arbitrary",)),
    )(tile_expert, nt_valid, xg, w13_f32, w2_bf, ws3)


def _shared_mlp_body(x_ref, wgu_ref, wd_ref, out_ref, wgu_bf, wd_bf):
    @pl.when(pl.program_id(0) == 0)
    def _():
        wgu_bf[...] = wgu_ref[...].astype(jnp.bfloat16)
        wd_bf[...] = wd_ref[...].astype(jnp.bfloat16)

    gate_up = jax.lax.dot_general(
        x_ref[...], wgu_bf[...],
        dimension_numbers=(((1,), (1,)), ((), ())),
        preferred_element_type=jnp.float32)
    gate, up = jnp.split(gate_up, 2, axis=-1)
    act = (jax.nn.silu(gate) * up).astype(x_ref.dtype)
    out_ref[...] = jax.lax.dot_general(
        act, wd_bf[...],
        dimension_numbers=(((1,), (1,)), ((), ())),
        preferred_element_type=jnp.float32)


def _shared_mlp(x_bf, sgu, sd, tile_t):
    t, h = x_bf.shape
    i2 = sgu.shape[0]
    i_dim = i2 // 2
    nt = t // tile_t
    return pl.pallas_call(
        _shared_mlp_body,
        grid=(nt,),
        in_specs=[
            pl.BlockSpec((tile_t, h), lambda i: (i, 0)),
            pl.BlockSpec((i2, h), lambda i: (0, 0)),
            pl.BlockSpec((h, i_dim), lambda i: (0, 0)),
        ],
        out_specs=pl.BlockSpec((tile_t, h), lambda i: (i, 0)),
        out_shape=jax.ShapeDtypeStruct((t, h), jnp.float32),
        scratch_shapes=[
            pltpu.VMEM((i2, h), jnp.bfloat16),
            pltpu.VMEM((h, i_dim), jnp.bfloat16),
        ],
    )(x_bf, sgu, sd)


def _routing_metadata(topk_idx, topk_weights, e, tile, maxtiles, pad_a):
    t, k = topk_idx.shape
    a = t * k
    ti = topk_idx.reshape(-1).astype(jnp.int32)
    onehot = (ti[:, None] == jnp.arange(e, dtype=jnp.int32)[None, :])
    onehot = onehot.astype(jnp.int32)
    counts = onehot.sum(axis=0)
    ranks_excl = jnp.cumsum(onehot, axis=0) - onehot
    rank = jnp.take_along_axis(ranks_excl, ti[:, None], axis=1)[:, 0]
    padded_counts = ((counts + tile - 1) // tile) * tile
    pad_off = jnp.concatenate(
        [jnp.zeros((1,), jnp.int32),
         jnp.cumsum(padded_counts)[:-1].astype(jnp.int32)])
    slot = pad_off[ti] + rank
    w_bits = jax.lax.bitcast_convert_type(
        topk_weights.reshape(-1).astype(jnp.float32), jnp.int32)
    vals = jnp.stack([jnp.arange(a, dtype=jnp.int32), w_bits], axis=1)
    packed = jnp.zeros((pad_a, 2), jnp.int32).at[slot].set(vals)
    inv_a = packed[:, 0]

    ntile_e = (padded_counts // tile).astype(jnp.int32)
    cum_tiles = jnp.cumsum(ntile_e)
    tile_expert = jnp.searchsorted(
        cum_tiles, jnp.arange(maxtiles, dtype=jnp.int32), side="right")
    tile_expert = jnp.minimum(tile_expert, e - 1).astype(jnp.int32)
    nt_valid = cum_tiles[-1:].astype(jnp.int32)

    s_arange = jnp.arange(pad_a, dtype=jnp.int32)
    slot_expert = jnp.repeat(tile_expert, tile)
    rank_in_e = s_arange - pad_off[slot_expert]
    valid = rank_in_e < counts[slot_expert]
    gtok = jnp.where(valid, inv_a // k, s_arange % t)
    wslot = jnp.where(
        valid, jax.lax.bitcast_convert_type(packed[:, 1], jnp.float32), 0.0)
    pos = slot.reshape(t, k)
    return gtok, wslot, pos, tile_expert, nt_valid


def kernel(hidden_states, topk_idx, topk_weights, w13, w2,
           shared_gate_up, shared_down):
    t, h = hidden_states.shape
    e, i2, _ = w13.shape
    k = topk_idx.shape[1]
    a = t * k
    tile = 128
    maxtiles = a // tile + e
    pad_a = maxtiles * tile

    x_bf = hidden_states.astype(jnp.bfloat16)
    shared_output = _shared_mlp(x_bf, shared_gate_up, shared_down,
                                min(256, t))
    w2_bf = w2.astype(jnp.bfloat16)

    gtok, wslot, pos, tile_expert, nt_valid = _routing_metadata(
        topk_idx, topk_weights, e, tile, maxtiles, pad_a)

    xg = _dispatch_sc(hidden_states, gtok, pad_a)
    dg = _grouped_gemm(xg, w13, w2_bf, wslot, tile_expert, nt_valid,
                       tile, maxtiles)
    combined_x = _combine_sc(dg, pos[:, 0], pos[:, 1])

    return (combined_x, shared_output)


# speedup vs baseline: 1.3167x; 1.3167x over previous
import functools

import jax
import jax.numpy as jnp
from jax import lax
from jax.experimental import pallas as pl
from jax.experimental.pallas import tpu as pltpu
from jax.experimental.pallas import tpu_sc as plsc

_NC = 2
_NS = 16
_NW = _NC * _NS
_L = 16


def _dispatch_sc(x, gtok, pad_a):
    t, h = x.shape
    b_per_w = pad_a // _NW
    ch = b_per_w // 4
    mesh = plsc.VectorSubcoreMesh(core_axis_name="c", subcore_axis_name="s")

    @functools.partial(
        pl.kernel, mesh=mesh,
        out_type=jax.ShapeDtypeStruct((pad_a, h), jnp.float32),
        scratch_types=[
            pltpu.VMEM((b_per_w,), jnp.int32),
            pltpu.VMEM((ch, h), jnp.float32),
            pltpu.VMEM((ch, h), jnp.float32),
            pltpu.SemaphoreType.DMA,
            pltpu.SemaphoreType.DMA,
            pltpu.SemaphoreType.DMA,
            pltpu.SemaphoreType.DMA,
        ],
    )
    def k(x_hbm, idx_hbm, out_hbm, idx_v, rows0, rows1, sg0, sg1, sw0, sw1):
        wid = lax.axis_index("s") * _NC + lax.axis_index("c")
        base = wid * b_per_w
        pltpu.sync_copy(idx_hbm.at[pl.ds(base, b_per_w)], idx_v)
        g0 = pltpu.async_copy(x_hbm.at[idx_v.at[pl.ds(0, ch)]], rows0, sg0)
        g1 = pltpu.async_copy(x_hbm.at[idx_v.at[pl.ds(ch, ch)]], rows1, sg1)
        g0.wait()
        w0 = pltpu.async_copy(rows0, out_hbm.at[pl.ds(base, ch)], sw0)
        g1.wait()
        w1 = pltpu.async_copy(rows1, out_hbm.at[pl.ds(base + ch, ch)], sw1)
        w0.wait()
        g2 = pltpu.async_copy(
            x_hbm.at[idx_v.at[pl.ds(2 * ch, ch)]], rows0, sg0)
        w1.wait()
        g3 = pltpu.async_copy(
            x_hbm.at[idx_v.at[pl.ds(3 * ch, ch)]], rows1, sg1)
        g2.wait()
        w2 = pltpu.async_copy(rows0, out_hbm.at[pl.ds(base + 2 * ch, ch)], sw0)
        g3.wait()
        w3 = pltpu.async_copy(rows1, out_hbm.at[pl.ds(base + 3 * ch, ch)], sw1)
        w2.wait()
        w3.wait()

    return k(x, gtok)


def _combine_sc(dg, pos0, pos1):
    t = pos0.shape[0]
    h = dg.shape[1]
    t_per_w = t // _NW
    cht = min(32, t_per_w)
    mesh = plsc.VectorSubcoreMesh(core_axis_name="c", subcore_axis_name="s")

    @functools.partial(
        pl.kernel, mesh=mesh,
        out_type=jax.ShapeDtypeStruct((t, h), jnp.float32),
        scratch_types=[
            pltpu.VMEM((t_per_w,), jnp.int32),
            pltpu.VMEM((t_per_w,), jnp.int32),
            pltpu.VMEM((cht, h), jnp.float32),
            pltpu.VMEM((cht, h), jnp.float32),
            pltpu.SemaphoreType.DMA,
        ],
    )
    def k(dg_hbm, p0_hbm, p1_hbm, out_hbm, p0_v, p1_v, buf0, buf1, sem):
        wid = lax.axis_index("s") * _NC + lax.axis_index("c")
        base = wid * t_per_w
        pltpu.sync_copy(p0_hbm.at[pl.ds(base, t_per_w)], p0_v)
        pltpu.sync_copy(p1_hbm.at[pl.ds(base, t_per_w)], p1_v)

        @pl.loop(0, t_per_w // cht)
        def _(c):
            cp0 = pltpu.async_copy(
                dg_hbm.at[p0_v.at[pl.ds(c * cht, cht)]], buf0, sem)
            cp1 = pltpu.async_copy(
                dg_hbm.at[p1_v.at[pl.ds(c * cht, cht)]], buf1, sem)
            cp0.wait()
            cp1.wait()

            @pl.loop(0, cht)
            def _(r):
                @pl.loop(0, h // (4 * _L))
                def _(q):
                    for u in range(4):
                        slc = (pl.ds(r, 1), pl.ds(q * 4 * _L + u * _L, _L))
                        buf0.at[*slc][...] = (
                            buf0.at[*slc][...] + buf1.at[*slc][...])

            pltpu.sync_copy(buf0, out_hbm.at[pl.ds(base + c * cht, cht)])

    return k(dg, pos0, pos1)


def _grouped_gemm_body(te_ref, nt_ref, xg_ref, w13_ref, w2_ref, ws_ref,
                       dg_ref, w13bf_ref):
    i = pl.program_id(0)
    i_dim = w13_ref.shape[1] // 2
    te_i = te_ref[i]
    te_prev = te_ref[jnp.maximum(i - 1, 0)]
    new_expert = jnp.logical_or(i == 0, te_i != te_prev)

    @pl.when(jnp.logical_and(i < nt_ref[0], new_expert))
    def _():
        w13bf_ref[...] = w13_ref[0].astype(jnp.bfloat16)

    @pl.when(i < nt_ref[0])
    def _():
        x = xg_ref[...].astype(jnp.bfloat16)
        gate = jax.lax.dot_general(
            x, w13bf_ref[:i_dim],
            dimension_numbers=(((1,), (1,)), ((), ())),
            preferred_element_type=jnp.float32)
        up = jax.lax.dot_general(
            x, w13bf_ref[i_dim:],
            dimension_numbers=(((1,), (1,)), ((), ())),
            preferred_element_type=jnp.float32)
        act = (jax.nn.silu(gate) * up).astype(jnp.bfloat16)
        down = jax.lax.dot_general(
            act, w2_ref[0],
            dimension_numbers=(((1,), (1,)), ((), ())),
            preferred_element_type=jnp.float32)
        dg_ref[...] = down * ws_ref[0]


def _grouped_gemm(xg, w13_f32, w2_bf, wslot, tile_expert, nt_valid,
                  tile, maxtiles):
    pad_a, h = xg.shape
    e, i2, _ = w13_f32.shape
    i_dim = i2 // 2
    ws3 = wslot.reshape(maxtiles, tile, 1)
    grid_spec = pltpu.PrefetchScalarGridSpec(
        num_scalar_prefetch=2,
        grid=(maxtiles,),
        in_specs=[
            pl.BlockSpec((tile, h), lambda i, te, nt: (i, 0)),
            pl.BlockSpec((1, i2, h), lambda i, te, nt: (te[i], 0, 0)),
            pl.BlockSpec((1, h, i_dim), lambda i, te, nt: (te[i], 0, 0)),
            pl.BlockSpec((1, tile, 1), lambda i, te, nt: (i, 0, 0)),
        ],
        out_specs=pl.BlockSpec((tile, h), lambda i, te, nt: (i, 0)),
        scratch_shapes=[pltpu.VMEM((i2, h), jnp.bfloat16)],
    )
    return pl.pallas_call(
        _grouped_gemm_body,
        grid_spec=grid_spec,
        out_shape=jax.ShapeDtypeStruct((pad_a, h), jnp.float32),
        compiler_params=pltpu.CompilerParams(
            dimension_semantics=("arbitrary",)),
    )(tile_expert, nt_valid, xg, w13_f32, w2_bf, ws3)


def _shared_mlp_body(x_ref, wgu_ref, wd_ref, out_ref, wgu_bf, wd_bf):
    @pl.when(pl.program_id(0) == 0)
    def _():
        wgu_bf[...] = wgu_ref[...].astype(jnp.bfloat16)
        wd_bf[...] = wd_ref[...].astype(jnp.bfloat16)

    gate_up = jax.lax.dot_general(
        x_ref[...], wgu_bf[...],
        dimension_numbers=(((1,), (1,)), ((), ())),
        preferred_element_type=jnp.float32)
    gate, up = jnp.split(gate_up, 2, axis=-1)
    act = (jax.nn.silu(gate) * up).astype(x_ref.dtype)
    out_ref[...] = jax.lax.dot_general(
        act, wd_bf[...],
        dimension_numbers=(((1,), (1,)), ((), ())),
        preferred_element_type=jnp.float32)


def _shared_mlp(x_bf, sgu, sd, tile_t):
    t, h = x_bf.shape
    i2 = sgu.shape[0]
    i_dim = i2 // 2
    nt = t // tile_t
    return pl.pallas_call(
        _shared_mlp_body,
        grid=(nt,),
        in_specs=[
            pl.BlockSpec((tile_t, h), lambda i: (i, 0)),
            pl.BlockSpec((i2, h), lambda i: (0, 0)),
            pl.BlockSpec((h, i_dim), lambda i: (0, 0)),
        ],
        out_specs=pl.BlockSpec((tile_t, h), lambda i: (i, 0)),
        out_shape=jax.ShapeDtypeStruct((t, h), jnp.float32),
        scratch_shapes=[
            pltpu.VMEM((i2, h), jnp.bfloat16),
            pltpu.VMEM((h, i_dim), jnp.bfloat16),
        ],
    )(x_bf, sgu, sd)


def _routing_metadata(topk_idx, topk_weights, e, tile, maxtiles, pad_a):
    t, k = topk_idx.shape
    a = t * k
    ti = topk_idx.reshape(-1).astype(jnp.int32)
    onehot = (ti[:, None] == jnp.arange(e, dtype=jnp.int32)[None, :])
    onehot = onehot.astype(jnp.int32)
    counts = onehot.sum(axis=0)
    ranks_excl = jnp.cumsum(onehot, axis=0) - onehot
    rank = jnp.take_along_axis(ranks_excl, ti[:, None], axis=1)[:, 0]
    padded_counts = ((counts + tile - 1) // tile) * tile
    pad_off = jnp.concatenate(
        [jnp.zeros((1,), jnp.int32),
         jnp.cumsum(padded_counts)[:-1].astype(jnp.int32)])
    slot = pad_off[ti] + rank
    w_bits = jax.lax.bitcast_convert_type(
        topk_weights.reshape(-1).astype(jnp.float32), jnp.int32)
    vals = jnp.stack([jnp.arange(a, dtype=jnp.int32), w_bits], axis=1)
    packed = jnp.zeros((pad_a, 2), jnp.int32).at[slot].set(vals)
    inv_a = packed[:, 0]

    ntile_e = (padded_counts // tile).astype(jnp.int32)
    cum_tiles = jnp.cumsum(ntile_e)
    tile_expert = jnp.searchsorted(
        cum_tiles, jnp.arange(maxtiles, dtype=jnp.int32), side="right")
    tile_expert = jnp.minimum(tile_expert, e - 1).astype(jnp.int32)
    nt_valid = cum_tiles[-1:].astype(jnp.int32)

    s_arange = jnp.arange(pad_a, dtype=jnp.int32)
    slot_expert = jnp.repeat(tile_expert, tile)
    rank_in_e = s_arange - pad_off[slot_expert]
    valid = rank_in_e < counts[slot_expert]
    gtok = jnp.where(valid, inv_a // k, s_arange % t)
    wslot = jnp.where(
        valid, jax.lax.bitcast_convert_type(packed[:, 1], jnp.float32), 0.0)
    pos = slot.reshape(t, k)
    return gtok, wslot, pos, tile_expert, nt_valid


def kernel(hidden_states, topk_idx, topk_weights, w13, w2,
           shared_gate_up, shared_down):
    t, h = hidden_states.shape
    e, i2, _ = w13.shape
    k = topk_idx.shape[1]
    a = t * k
    tile = 256
    maxtiles = a // tile + e
    pad_a = maxtiles * tile

    x_bf = hidden_states.astype(jnp.bfloat16)
    shared_output = _shared_mlp(x_bf, shared_gate_up, shared_down,
                                min(256, t))
    w2_bf = w2.astype(jnp.bfloat16)

    gtok, wslot, pos, tile_expert, nt_valid = _routing_metadata(
        topk_idx, topk_weights, e, tile, maxtiles, pad_a)

    xg = _dispatch_sc(hidden_states, gtok, pad_a)
    dg = _grouped_gemm(xg, w13, w2_bf, wslot, tile_expert, nt_valid,
                       tile, maxtiles)
    combined_x = _combine_sc(dg, pos[:, 0], pos[:, 1])

    return (combined_x, shared_output)


# speedup vs baseline: 1.3410x; 1.0185x over previous
import functools

import jax
import jax.numpy as jnp
from jax import lax
from jax.experimental import pallas as pl
from jax.experimental.pallas import tpu as pltpu
from jax.experimental.pallas import tpu_sc as plsc

_NC = 2
_NS = 16
_NW = _NC * _NS
_L = 16


def _dispatch_sc(x, gtok, pad_a):
    t, h = x.shape
    b_per_w = pad_a // _NW
    ch = b_per_w // 4
    mesh = plsc.VectorSubcoreMesh(core_axis_name="c", subcore_axis_name="s")

    @functools.partial(
        pl.kernel, mesh=mesh,
        out_type=jax.ShapeDtypeStruct((pad_a, h), jnp.float32),
        scratch_types=[
            pltpu.VMEM((b_per_w,), jnp.int32),
            pltpu.VMEM((ch, h), jnp.float32),
            pltpu.VMEM((ch, h), jnp.float32),
            pltpu.SemaphoreType.DMA,
            pltpu.SemaphoreType.DMA,
            pltpu.SemaphoreType.DMA,
            pltpu.SemaphoreType.DMA,
        ],
    )
    def k(x_hbm, idx_hbm, out_hbm, idx_v, rows0, rows1, sg0, sg1, sw0, sw1):
        wid = lax.axis_index("s") * _NC + lax.axis_index("c")
        base = wid * b_per_w
        pltpu.sync_copy(idx_hbm.at[pl.ds(base, b_per_w)], idx_v)
        g0 = pltpu.async_copy(x_hbm.at[idx_v.at[pl.ds(0, ch)]], rows0, sg0)
        g1 = pltpu.async_copy(x_hbm.at[idx_v.at[pl.ds(ch, ch)]], rows1, sg1)
        g0.wait()
        w0 = pltpu.async_copy(rows0, out_hbm.at[pl.ds(base, ch)], sw0)
        g1.wait()
        w1 = pltpu.async_copy(rows1, out_hbm.at[pl.ds(base + ch, ch)], sw1)
        w0.wait()
        g2 = pltpu.async_copy(
            x_hbm.at[idx_v.at[pl.ds(2 * ch, ch)]], rows0, sg0)
        w1.wait()
        g3 = pltpu.async_copy(
            x_hbm.at[idx_v.at[pl.ds(3 * ch, ch)]], rows1, sg1)
        g2.wait()
        w2 = pltpu.async_copy(rows0, out_hbm.at[pl.ds(base + 2 * ch, ch)], sw0)
        g3.wait()
        w3 = pltpu.async_copy(rows1, out_hbm.at[pl.ds(base + 3 * ch, ch)], sw1)
        w2.wait()
        w3.wait()

    return k(x, gtok)


def _combine_sc(dg, pos0, pos1):
    t = pos0.shape[0]
    h = dg.shape[1]
    t_per_w = t // _NW
    cht = min(32, t_per_w)
    mesh = plsc.VectorSubcoreMesh(core_axis_name="c", subcore_axis_name="s")

    @functools.partial(
        pl.kernel, mesh=mesh,
        out_type=jax.ShapeDtypeStruct((t, h), jnp.float32),
        scratch_types=[
            pltpu.VMEM((t_per_w,), jnp.int32),
            pltpu.VMEM((t_per_w,), jnp.int32),
            pltpu.VMEM((cht, h), jnp.float32),
            pltpu.VMEM((cht, h), jnp.float32),
            pltpu.SemaphoreType.DMA,
        ],
    )
    def k(dg_hbm, p0_hbm, p1_hbm, out_hbm, p0_v, p1_v, buf0, buf1, sem):
        wid = lax.axis_index("s") * _NC + lax.axis_index("c")
        base = wid * t_per_w
        pltpu.sync_copy(p0_hbm.at[pl.ds(base, t_per_w)], p0_v)
        pltpu.sync_copy(p1_hbm.at[pl.ds(base, t_per_w)], p1_v)

        @pl.loop(0, t_per_w // cht)
        def _(c):
            cp0 = pltpu.async_copy(
                dg_hbm.at[p0_v.at[pl.ds(c * cht, cht)]], buf0, sem)
            cp1 = pltpu.async_copy(
                dg_hbm.at[p1_v.at[pl.ds(c * cht, cht)]], buf1, sem)
            cp0.wait()
            cp1.wait()

            @pl.loop(0, cht)
            def _(r):
                @pl.loop(0, h // (4 * _L))
                def _(q):
                    for u in range(4):
                        slc = (pl.ds(r, 1), pl.ds(q * 4 * _L + u * _L, _L))
                        buf0.at[*slc][...] = (
                            buf0.at[*slc][...] + buf1.at[*slc][...])

            pltpu.sync_copy(buf0, out_hbm.at[pl.ds(base + c * cht, cht)])

    return k(dg, pos0, pos1)


def _grouped_gemm_body(te_ref, nt_ref, xg_ref, w13_ref, w2_ref, ws_ref,
                       dg_ref):
    i = pl.program_id(0)
    i_dim = w13_ref.shape[1] // 2

    @pl.when(i < nt_ref[0])
    def _():
        x = xg_ref[...].astype(jnp.bfloat16)
        gate = jax.lax.dot_general(
            x, w13_ref[0, :i_dim].astype(jnp.bfloat16),
            dimension_numbers=(((1,), (1,)), ((), ())),
            preferred_element_type=jnp.float32)
        up = jax.lax.dot_general(
            x, w13_ref[0, i_dim:].astype(jnp.bfloat16),
            dimension_numbers=(((1,), (1,)), ((), ())),
            preferred_element_type=jnp.float32)
        act = (jax.nn.silu(gate) * up).astype(jnp.bfloat16)
        down = jax.lax.dot_general(
            act, w2_ref[0],
            dimension_numbers=(((1,), (1,)), ((), ())),
            preferred_element_type=jnp.float32)
        dg_ref[...] = down * ws_ref[0]


def _grouped_gemm(xg, w13_f32, w2_bf, wslot, tile_expert, nt_valid,
                  tile, maxtiles):
    pad_a, h = xg.shape
    e, i2, _ = w13_f32.shape
    i_dim = i2 // 2
    ws3 = wslot.reshape(maxtiles, tile, 1)
    grid_spec = pltpu.PrefetchScalarGridSpec(
        num_scalar_prefetch=2,
        grid=(maxtiles,),
        in_specs=[
            pl.BlockSpec((tile, h), lambda i, te, nt: (i, 0)),
            pl.BlockSpec((1, i2, h), lambda i, te, nt: (te[i], 0, 0)),
            pl.BlockSpec((1, h, i_dim), lambda i, te, nt: (te[i], 0, 0)),
            pl.BlockSpec((1, tile, 1), lambda i, te, nt: (i, 0, 0)),
        ],
        out_specs=pl.BlockSpec((tile, h), lambda i, te, nt: (i, 0)),
    )
    return pl.pallas_call(
        _grouped_gemm_body,
        grid_spec=grid_spec,
        out_shape=jax.ShapeDtypeStruct((pad_a, h), jnp.float32),
        compiler_params=pltpu.CompilerParams(
            dimension_semantics=("arbitrary",)),
    )(tile_expert, nt_valid, xg, w13_f32, w2_bf, ws3)


def _shared_mlp_body(x_ref, wgu_ref, wd_ref, out_ref, wgu_bf, wd_bf):
    @pl.when(pl.program_id(0) == 0)
    def _():
        wgu_bf[...] = wgu_ref[...].astype(jnp.bfloat16)
        wd_bf[...] = wd_ref[...].astype(jnp.bfloat16)

    gate_up = jax.lax.dot_general(
        x_ref[...], wgu_bf[...],
        dimension_numbers=(((1,), (1,)), ((), ())),
        preferred_element_type=jnp.float32)
    gate, up = jnp.split(gate_up, 2, axis=-1)
    act = (jax.nn.silu(gate) * up).astype(x_ref.dtype)
    out_ref[...] = jax.lax.dot_general(
        act, wd_bf[...],
        dimension_numbers=(((1,), (1,)), ((), ())),
        preferred_element_type=jnp.float32)


def _shared_mlp(x_bf, sgu, sd, tile_t):
    t, h = x_bf.shape
    i2 = sgu.shape[0]
    i_dim = i2 // 2
    nt = t // tile_t
    return pl.pallas_call(
        _shared_mlp_body,
        grid=(nt,),
        in_specs=[
            pl.BlockSpec((tile_t, h), lambda i: (i, 0)),
            pl.BlockSpec((i2, h), lambda i: (0, 0)),
            pl.BlockSpec((h, i_dim), lambda i: (0, 0)),
        ],
        out_specs=pl.BlockSpec((tile_t, h), lambda i: (i, 0)),
        out_shape=jax.ShapeDtypeStruct((t, h), jnp.float32),
        scratch_shapes=[
            pltpu.VMEM((i2, h), jnp.bfloat16),
            pltpu.VMEM((h, i_dim), jnp.bfloat16),
        ],
    )(x_bf, sgu, sd)


def _routing_metadata(topk_idx, topk_weights, e, tile, maxtiles, pad_a):
    t, k = topk_idx.shape
    a = t * k
    ti = topk_idx.reshape(-1).astype(jnp.int32)
    onehot = (ti[:, None] == jnp.arange(e, dtype=jnp.int32)[None, :])
    onehot = onehot.astype(jnp.int32)
    counts = onehot.sum(axis=0)
    ranks_excl = jnp.cumsum(onehot, axis=0) - onehot
    rank = jnp.take_along_axis(ranks_excl, ti[:, None], axis=1)[:, 0]
    padded_counts = ((counts + tile - 1) // tile) * tile
    pad_off = jnp.concatenate(
        [jnp.zeros((1,), jnp.int32),
         jnp.cumsum(padded_counts)[:-1].astype(jnp.int32)])
    slot = pad_off[ti] + rank
    w_bits = jax.lax.bitcast_convert_type(
        topk_weights.reshape(-1).astype(jnp.float32), jnp.int32)
    vals = jnp.stack([jnp.arange(a, dtype=jnp.int32), w_bits], axis=1)
    packed = jnp.zeros((pad_a, 2), jnp.int32).at[slot].set(vals)
    inv_a = packed[:, 0]

    ntile_e = (padded_counts // tile).astype(jnp.int32)
    cum_tiles = jnp.cumsum(ntile_e)
    tile_expert = jnp.searchsorted(
        cum_tiles, jnp.arange(maxtiles, dtype=jnp.int32), side="right")
    tile_expert = jnp.minimum(tile_expert, e - 1).astype(jnp.int32)
    nt_valid = cum_tiles[-1:].astype(jnp.int32)

    s_arange = jnp.arange(pad_a, dtype=jnp.int32)
    slot_expert = jnp.repeat(tile_expert, tile)
    rank_in_e = s_arange - pad_off[slot_expert]
    valid = rank_in_e < counts[slot_expert]
    gtok = jnp.where(valid, inv_a // k, s_arange % t)
    wslot = jnp.where(
        valid, jax.lax.bitcast_convert_type(packed[:, 1], jnp.float32), 0.0)
    pos = slot.reshape(t, k)
    return gtok, wslot, pos, tile_expert, nt_valid


def kernel(hidden_states, topk_idx, topk_weights, w13, w2,
           shared_gate_up, shared_down):
    t, h = hidden_states.shape
    e, i2, _ = w13.shape
    k = topk_idx.shape[1]
    a = t * k
    tile = 256
    maxtiles = a // tile + e
    pad_a = maxtiles * tile

    x_bf = hidden_states.astype(jnp.bfloat16)
    shared_output = _shared_mlp(x_bf, shared_gate_up, shared_down,
                                min(256, t))
    w2_bf = w2.astype(jnp.bfloat16)

    gtok, wslot, pos, tile_expert, nt_valid = _routing_metadata(
        topk_idx, topk_weights, e, tile, maxtiles, pad_a)

    xg = _dispatch_sc(hidden_states, gtok, pad_a)
    dg = _grouped_gemm(xg, w13, w2_bf, wslot, tile_expert, nt_valid,
                       tile, maxtiles)
    combined_x = _combine_sc(dg, pos[:, 0], pos[:, 1])

    return (combined_x, shared_output)
